# Initial kernel scaffold; baseline (speedup 1.0000x reference)
import jax
import jax.numpy as jnp
from jax.experimental import pallas as pl


def kernel(x, edge_index, edge_attr, batch, w0_rel, b0, w0_root, w1_rel, b1, w1_root, w2_rel, b2, w2_root, w3_rel, b3, w3_root, w4_rel, b4, w4_root, w5_rel, b5, w5_root):
    raise NotImplementedError("write your pallas kernel here")


# speedup vs baseline: 2.4126x; 2.4126x over previous
import functools

import jax
import jax.numpy as jnp
from jax import lax
from jax.experimental import pallas as pl
from jax.experimental.pallas import tpu as pltpu
from jax.experimental.pallas import tpu_sc as plsc

N = 10000
E = 320000
G = 16
NPAD = 10240
TRASH = N
MBLK = 256
NBLK = NPAD // MBLK
EB = 128
EPAD = 327680
TPR = NPAD // 16
HP = 2048
HC = HP // 128
F32 = jnp.float32


def _seg_mesh():
    return plsc.VectorSubcoreMesh(core_axis_name="c", subcore_axis_name="s",
                                  num_cores=2, num_subcores=16)


def _seg128(table, srcl, dstl, zrows):
    ept = EPAD // 32
    nb = ept // EB

    @functools.partial(
        pl.kernel,
        mesh=_seg_mesh(),
        out_type=jax.ShapeDtypeStruct((2 * NPAD, 128), F32),
        scratch_types=[
            pltpu.VMEM((EB,), jnp.int32),
            pltpu.VMEM((EB,), jnp.int32),
            pltpu.VMEM((EB, 128), F32),
            pltpu.VMEM_SHARED((NPAD, 128), F32),
            pltpu.SemaphoreType.DMA,
        ],
    )
    def k(table_r, src_r, dst_r, z_r, out_r, sidx, didx, rows, shared, sem):
        c = lax.axis_index("c")
        s = lax.axis_index("s")
        g = c * 16 + s
        pltpu.sync_copy(z_r, shared.at[pl.ds(s * TPR, TPR)])
        plsc.subcore_barrier()
        base = g * ept

        def body(i, carry):
            off = base + i * EB
            pltpu.sync_copy(src_r.at[pl.ds(off, EB)], sidx)
            pltpu.sync_copy(dst_r.at[pl.ds(off, EB)], didx)
            pltpu.async_copy(table_r.at[sidx], rows, sem).wait()
            pltpu.sync_copy(rows, shared.at[didx], add=True)
            return carry

        lax.fori_loop(0, nb, body, 0)
        plsc.subcore_barrier()
        pltpu.sync_copy(
            shared.at[pl.ds(s * TPR, TPR)],
            out_r.at[pl.ds(c * NPAD + s * TPR, TPR)],
        )

    return k(table, srcl, dstl, zrows).reshape(2, NPAD, 128)


def _segchunk(table, src_all, dstl, zrows):
    ept = EPAD // 16
    nb = ept // EB

    @functools.partial(
        pl.kernel,
        mesh=_seg_mesh(),
        out_type=jax.ShapeDtypeStruct((HC * NPAD, 128), F32),
        scratch_types=[
            pltpu.VMEM((EB,), jnp.int32),
            pltpu.VMEM((EB,), jnp.int32),
            pltpu.VMEM((EB, 128), F32),
            pltpu.VMEM_SHARED((NPAD, 128), F32),
            pltpu.SemaphoreType.DMA,
        ],
    )
    def k(table_r, src_r, dst_r, z_r, out_r, sidx, didx, rows, shared, sem):
        c = lax.axis_index("c")
        s = lax.axis_index("s")

        def chunk_body(j, carry):
            ch = c * 8 + j
            pltpu.sync_copy(z_r, shared.at[pl.ds(s * TPR, TPR)])
            plsc.subcore_barrier()

            def body(i, icarry):
                off = s * ept + i * EB
                pltpu.sync_copy(src_r.at[pl.ds(ch * EPAD + off, EB)], sidx)
                pltpu.sync_copy(dst_r.at[pl.ds(off, EB)], didx)
                pltpu.async_copy(table_r.at[sidx], rows, sem).wait()
                pltpu.sync_copy(rows, shared.at[didx], add=True)
                return icarry

            lax.fori_loop(0, nb, body, 0)
            plsc.subcore_barrier()
            pltpu.sync_copy(
                shared.at[pl.ds(s * TPR, TPR)],
                out_r.at[pl.ds(ch * NPAD + s * TPR, TPR)],
            )
            return carry

        lax.fori_loop(0, 8, chunk_body, 0)

    return k(table, src_all, dstl, zrows).reshape(HC, NPAD, 128)


def _mm2(a1, w1, a2, w2, bias, relu):
    k1 = a1.shape[0]
    k2 = a2.shape[0]
    nk = k1 + k2
    m_out = w1.shape[1]
    oc = m_out // 128

    def body(a1_ref, w1_ref, a2_ref, w2_ref, b_ref, o_ref, acc):
        ki = pl.program_id(1)

        @pl.when(ki == 0)
        def _():
            acc[...] = jnp.zeros_like(acc)

        @pl.when(ki < k1)
        def _():
            acc[...] += jnp.dot(a1_ref[0], w1_ref[...],
                                preferred_element_type=F32)

        @pl.when(ki >= k1)
        def _():
            acc[...] += jnp.dot(a2_ref[0], w2_ref[...],
                                preferred_element_type=F32)

        @pl.when(ki == nk - 1)
        def _():
            z = acc[...] + b_ref[...]
            if relu:
                z = jnp.maximum(z, 0.0)
            for ci in range(oc):
                o_ref[ci] = z[:, ci * 128:(ci + 1) * 128]

    return pl.pallas_call(
        body,
        grid=(NBLK, nk),
        in_specs=[
            pl.BlockSpec((1, MBLK, 128), lambda m, k: (jnp.minimum(k, k1 - 1), m, 0)),
            pl.BlockSpec((128, m_out), lambda m, k: (jnp.minimum(k, k1 - 1), 0)),
            pl.BlockSpec((1, MBLK, 128), lambda m, k: (jnp.maximum(k - k1, 0), m, 0)),
            pl.BlockSpec((128, m_out), lambda m, k: (jnp.maximum(k - k1, 0), 0)),
            pl.BlockSpec((1, m_out), lambda m, k: (0, 0)),
        ],
        out_specs=pl.BlockSpec((oc, MBLK, 128), lambda m, k: (0, m, 0)),
        out_shape=jax.ShapeDtypeStruct((oc, NPAD, 128), F32),
        scratch_shapes=[pltpu.VMEM((MBLK, m_out), F32)],
        compiler_params=pltpu.CompilerParams(
            dimension_semantics=("parallel", "arbitrary")),
    )(a1, w1, a2, w2, bias)


def _mm1(a1, w1, bias, relu):
    nk = a1.shape[0]
    m_out = w1.shape[1]
    oc = m_out // 128

    def body(a1_ref, w1_ref, b_ref, o_ref, acc):
        ki = pl.program_id(1)

        @pl.when(ki == 0)
        def _():
            acc[...] = jnp.zeros_like(acc)

        acc[...] += jnp.dot(a1_ref[0], w1_ref[...], preferred_element_type=F32)

        @pl.when(ki == nk - 1)
        def _():
            z = acc[...] + b_ref[...]
            if relu:
                z = jnp.maximum(z, 0.0)
            for ci in range(oc):
                o_ref[ci] = z[:, ci * 128:(ci + 1) * 128]

    return pl.pallas_call(
        body,
        grid=(NBLK, nk),
        in_specs=[
            pl.BlockSpec((1, MBLK, 128), lambda m, k: (k, m, 0)),
            pl.BlockSpec((128, m_out), lambda m, k: (k, 0)),
            pl.BlockSpec((1, m_out), lambda m, k: (0, 0)),
        ],
        out_specs=pl.BlockSpec((oc, MBLK, 128), lambda m, k: (0, m, 0)),
        out_shape=jax.ShapeDtypeStruct((oc, NPAD, 128), F32),
        scratch_shapes=[pltpu.VMEM((MBLK, m_out), F32)],
        compiler_params=pltpu.CompilerParams(
            dimension_semantics=("parallel", "arbitrary")),
    )(a1, w1, bias)


def _h2enc(a2, y2r2, batch3):
    def body(a_ref, r_ref, bt_ref, h_ref, e_ref, hs, cs):
        m = pl.program_id(0)

        @pl.when(m == 0)
        def _():
            hs[...] = jnp.zeros_like(hs)
            cs[...] = jnp.zeros_like(cs)

        h = jnp.maximum(a_ref[0] + a_ref[1] + r_ref[0], 0.0)
        h_ref[...] = h
        bt = bt_ref[0, 0]
        grp = lax.broadcasted_iota(jnp.int32, (G, MBLK), 0)
        onehot = (bt[None, :] == grp).astype(F32)
        hs[...] += jnp.dot(onehot, h, preferred_element_type=F32)
        cnt = jnp.sum(onehot, axis=1, keepdims=True)
        cs[...] += jnp.broadcast_to(cnt, (G, 128))

        @pl.when(m == NBLK - 1)
        def _():
            e_ref[...] = hs[...] / jnp.maximum(cs[...], 1.0)

    return pl.pallas_call(
        body,
        grid=(NBLK,),
        in_specs=[
            pl.BlockSpec((2, MBLK, 128), lambda m: (0, m, 0)),
            pl.BlockSpec((1, MBLK, 128), lambda m: (1, m, 0)),
            pl.BlockSpec((1, 1, MBLK), lambda m: (m, 0, 0)),
        ],
        out_specs=[
            pl.BlockSpec((MBLK, 128), lambda m: (m, 0)),
            pl.BlockSpec((G, 128), lambda m: (0, 0)),
        ],
        out_shape=[
            jax.ShapeDtypeStruct((NPAD, 128), F32),
            jax.ShapeDtypeStruct((G, 128), F32),
        ],
        scratch_shapes=[pltpu.VMEM((G, 128), F32), pltpu.VMEM((G, 128), F32)],
        compiler_params=pltpu.CompilerParams(
            dimension_semantics=("arbitrary",)),
    )(a2, y2r2, batch3)


def _final(a5, y5r5):
    def body(a_ref, r_ref, o_ref):
        o_ref[...] = a_ref[0] + a_ref[1] + r_ref[0]

    return pl.pallas_call(
        body,
        grid=(NBLK,),
        in_specs=[
            pl.BlockSpec((2, MBLK, 128), lambda m: (0, m, 0)),
            pl.BlockSpec((1, MBLK, 128), lambda m: (1, m, 0)),
        ],
        out_specs=pl.BlockSpec((MBLK, 128), lambda m: (m, 0)),
        out_shape=jax.ShapeDtypeStruct((NPAD, 128), F32),
    )(a5, y5r5)


def _padw(w, r, c):
    return jnp.pad(w, ((0, r - w.shape[0]), (0, c - w.shape[1])))


def _padb(b, m_out):
    return jnp.pad(b, (0, m_out - b.shape[0])).reshape(1, m_out)


def kernel(x, edge_index, edge_attr, batch,
           w0_rel, b0, w0_root, w1_rel, b1, w1_root, w2_rel, b2, w2_root,
           w3_rel, b3, w3_root, w4_rel, b4, w4_root, w5_rel, b5, w5_root):
    src = edge_index[0]
    dst = edge_index[1]
    pad_e = EPAD - E
    src_p = jnp.concatenate([src, jnp.zeros((pad_e,), jnp.int32)])
    dst_p = jnp.concatenate([dst, jnp.full((pad_e,), TRASH, jnp.int32)])
    src_all = (src_p[None, :]
               + (jnp.arange(HC, dtype=jnp.int32) * NPAD)[:, None]).reshape(-1)
    batch3 = jnp.concatenate(
        [batch, jnp.full((NPAD - N,), G, jnp.int32)]).reshape(NBLK, 1, MBLK)
    xp = jnp.pad(x, ((0, NPAD - N), (0, 0)))
    zrows = jnp.zeros((TPR, 128), F32)

    w0a = _padw(jnp.concatenate([w0_rel, w0_rel], axis=0), 256, HP)
    w0b = _padw(w0_root, 128, HP)
    bias0 = _padb(b0, HP)
    w1a = _padw(w1_rel, HP, HP)
    w1b = _padw(w1_root, HP, HP)
    bias1 = _padb(b1, HP)
    w2c = _padw(jnp.concatenate([w2_rel, w2_root], axis=1), HP, 256)
    bias2 = jnp.concatenate([jnp.zeros((128,), F32), b2]).reshape(1, 256)
    w3a = _padw(jnp.concatenate([w3_rel, w3_rel], axis=0), 256, HP)
    w3b = _padw(w3_root, 128, HP)
    bias3 = _padb(b3, HP)
    w4a = _padw(w4_rel, HP, HP)
    w4b = _padw(w4_root, HP, HP)
    bias4 = _padb(b4, HP)
    w5c = _padw(jnp.concatenate([w5_rel, w5_root], axis=1), HP, 256)
    bias5 = jnp.concatenate([jnp.zeros((128,), F32), b5]).reshape(1, 256)

    a0 = _seg128(xp, src_p, dst_p, zrows)
    h0 = _mm2(a0, w0a, xp.reshape(1, NPAD, 128), w0b, bias0, True)
    a1 = _segchunk(h0.reshape(HC * NPAD, 128), src_all, dst_p, zrows)
    h1 = _mm2(a1, w1a, h0, w1b, bias1, True)
    y2r2 = _mm1(h1, w2c, bias2, False)
    a2 = _seg128(y2r2.reshape(2 * NPAD, 128), src_p, dst_p, zrows)
    h2, enc = _h2enc(a2, y2r2, batch3)
    a3 = _seg128(h2, src_p, dst_p, zrows)
    h3 = _mm2(a3, w3a, h2.reshape(1, NPAD, 128), w3b, bias3, True)
    a4 = _segchunk(h3.reshape(HC * NPAD, 128), src_all, dst_p, zrows)
    h4 = _mm2(a4, w4a, h3, w4b, bias4, True)
    y5r5 = _mm1(h4, w5c, bias5, False)
    a5 = _seg128(y5r5.reshape(2 * NPAD, 128), src_p, dst_p, zrows)
    out = _final(a5, y5r5)
    return (out[:N], enc)


# speedup vs baseline: 3.1008x; 1.2852x over previous
import functools

import jax
import jax.numpy as jnp
from jax import lax
from jax.experimental import pallas as pl
from jax.experimental.pallas import tpu as pltpu
from jax.experimental.pallas import tpu_sc as plsc

N = 10000
E = 320000
G = 16
NPAD = 10240
TRASH = N
MBLK = 256
NBLK = NPAD // MBLK
EB = 128
EPAD = 327680
TPR = NPAD // 16
HP = 2048
HC = HP // 128
F32 = jnp.float32


def _seg_mesh():
    return plsc.VectorSubcoreMesh(core_axis_name="c", subcore_axis_name="s",
                                  num_cores=2, num_subcores=16)


NRING = 2
IB = 40


def _edge_pipeline(table_r, shared, src_hbm, src_base, dst_hbm, dst_base,
                   src_v, dst_v, rows, gsems, ssems, nb):
    def gather(i, b):
        pltpu.async_copy(table_r.at[src_v.at[i]], rows.at[b], gsems[b])

    def wait_gather(b):
        pltpu.make_async_copy(table_r.at[src_v.at[0]], rows.at[b],
                              gsems[b]).wait()

    def scatter(i, b):
        pltpu.async_copy(rows.at[b], shared.at[dst_v.at[i]], ssems[b],
                         add=True)

    def wait_scatter(b):
        pltpu.make_async_copy(rows.at[b], shared.at[dst_v.at[0]],
                              ssems[b]).wait()

    def block(j2, carry):
        pltpu.sync_copy(src_hbm.at[pl.ds(src_base + j2 * IB, IB)], src_v)
        pltpu.sync_copy(dst_hbm.at[pl.ds(dst_base + j2 * IB, IB)], dst_v)
        for b in range(NRING):
            gather(b, b)

        def body(i, c2):
            for b in range(NRING):
                wait_gather(b)
                scatter(i + b, b)
            for b in range(NRING):
                wait_scatter(b)

                @pl.when(i + NRING + b < IB)
                def _():
                    gather(i + NRING + b, b)

            return c2

        lax.fori_loop(0, IB // NRING, lambda i2, c2: body(i2 * NRING, c2), 0)
        return carry

    lax.fori_loop(0, nb // IB, block, 0)


def _seg128(table, src2d, dst2d, zrows):
    ept = EPAD // 32
    nb = ept // EB

    @functools.partial(
        pl.kernel,
        mesh=_seg_mesh(),
        out_type=jax.ShapeDtypeStruct((2 * NPAD, 128), F32),
        scratch_types=[
            pltpu.VMEM((IB, EB), jnp.int32),
            pltpu.VMEM((IB, EB), jnp.int32),
            pltpu.VMEM((NRING, EB, 128), F32),
            pltpu.VMEM_SHARED((NPAD, 128), F32),
        ] + [pltpu.SemaphoreType.DMA] * (2 * NRING),
    )
    def k(table_r, src_r, dst_r, z_r, out_r, src_v, dst_v, rows, shared,
          *sems):
        c = lax.axis_index("c")
        s = lax.axis_index("s")
        g = c * 16 + s
        pltpu.sync_copy(z_r, shared.at[pl.ds(s * TPR, TPR)])
        plsc.subcore_barrier()
        _edge_pipeline(table_r, shared, src_r, g * nb, dst_r, g * nb,
                       src_v, dst_v, rows, sems[:NRING], sems[NRING:], nb)
        plsc.subcore_barrier()
        pltpu.sync_copy(
            shared.at[pl.ds(s * TPR, TPR)],
            out_r.at[pl.ds(c * NPAD + s * TPR, TPR)],
        )

    return k(table, src2d, dst2d, zrows).reshape(2, NPAD, 128)


def _segchunk(table, src_all, dst2d, zrows):
    ept = EPAD // 16
    nb = ept // EB
    nbt = EPAD // EB

    @functools.partial(
        pl.kernel,
        mesh=_seg_mesh(),
        out_type=jax.ShapeDtypeStruct((HC * NPAD, 128), F32),
        scratch_types=[
            pltpu.VMEM((IB, EB), jnp.int32),
            pltpu.VMEM((IB, EB), jnp.int32),
            pltpu.VMEM((NRING, EB, 128), F32),
            pltpu.VMEM_SHARED((NPAD, 128), F32),
        ] + [pltpu.SemaphoreType.DMA] * (2 * NRING),
    )
    def k(table_r, src_r, dst_r, z_r, out_r, src_v, dst_v, rows, shared,
          *sems):
        c = lax.axis_index("c")
        s = lax.axis_index("s")

        def chunk_body(j, carry):
            ch = c * 8 + j
            pltpu.sync_copy(z_r, shared.at[pl.ds(s * TPR, TPR)])
            plsc.subcore_barrier()
            _edge_pipeline(table_r, shared, src_r, ch * nbt + s * nb,
                           dst_r, s * nb, src_v, dst_v, rows,
                           sems[:NRING], sems[NRING:], nb)
            plsc.subcore_barrier()
            pltpu.sync_copy(
                shared.at[pl.ds(s * TPR, TPR)],
                out_r.at[pl.ds(ch * NPAD + s * TPR, TPR)],
            )
            return carry

        lax.fori_loop(0, 8, chunk_body, 0)

    return k(table, src_all, dst2d, zrows).reshape(HC, NPAD, 128)


def _mm2(a1, w1, a2, w2, bias, relu):
    k1 = a1.shape[0]
    k2 = a2.shape[0]
    nk = k1 + k2
    m_out = w1.shape[1]
    oc = m_out // 128

    def body(a1_ref, w1_ref, a2_ref, w2_ref, b_ref, o_ref, acc):
        ki = pl.program_id(1)

        @pl.when(ki == 0)
        def _():
            acc[...] = jnp.zeros_like(acc)

        @pl.when(ki < k1)
        def _():
            acc[...] += jnp.dot(a1_ref[0], w1_ref[...],
                                preferred_element_type=F32)

        @pl.when(ki >= k1)
        def _():
            acc[...] += jnp.dot(a2_ref[0], w2_ref[...],
                                preferred_element_type=F32)

        @pl.when(ki == nk - 1)
        def _():
            z = acc[...] + b_ref[...]
            if relu:
                z = jnp.maximum(z, 0.0)
            for ci in range(oc):
                o_ref[ci] = z[:, ci * 128:(ci + 1) * 128]

    return pl.pallas_call(
        body,
        grid=(NBLK, nk),
        in_specs=[
            pl.BlockSpec((1, MBLK, 128), lambda m, k: (jnp.minimum(k, k1 - 1), m, 0)),
            pl.BlockSpec((128, m_out), lambda m, k: (jnp.minimum(k, k1 - 1), 0)),
            pl.BlockSpec((1, MBLK, 128), lambda m, k: (jnp.maximum(k - k1, 0), m, 0)),
            pl.BlockSpec((128, m_out), lambda m, k: (jnp.maximum(k - k1, 0), 0)),
            pl.BlockSpec((1, m_out), lambda m, k: (0, 0)),
        ],
        out_specs=pl.BlockSpec((oc, MBLK, 128), lambda m, k: (0, m, 0)),
        out_shape=jax.ShapeDtypeStruct((oc, NPAD, 128), F32),
        scratch_shapes=[pltpu.VMEM((MBLK, m_out), F32)],
        compiler_params=pltpu.CompilerParams(
            dimension_semantics=("parallel", "arbitrary")),
    )(a1, w1, a2, w2, bias)


def _mm1(a1, w1, bias, relu):
    nk = a1.shape[0]
    m_out = w1.shape[1]
    oc = m_out // 128

    def body(a1_ref, w1_ref, b_ref, o_ref, acc):
        ki = pl.program_id(1)

        @pl.when(ki == 0)
        def _():
            acc[...] = jnp.zeros_like(acc)

        acc[...] += jnp.dot(a1_ref[0], w1_ref[...], preferred_element_type=F32)

        @pl.when(ki == nk - 1)
        def _():
            z = acc[...] + b_ref[...]
            if relu:
                z = jnp.maximum(z, 0.0)
            for ci in range(oc):
                o_ref[ci] = z[:, ci * 128:(ci + 1) * 128]

    return pl.pallas_call(
        body,
        grid=(NBLK, nk),
        in_specs=[
            pl.BlockSpec((1, MBLK, 128), lambda m, k: (k, m, 0)),
            pl.BlockSpec((128, m_out), lambda m, k: (k, 0)),
            pl.BlockSpec((1, m_out), lambda m, k: (0, 0)),
        ],
        out_specs=pl.BlockSpec((oc, MBLK, 128), lambda m, k: (0, m, 0)),
        out_shape=jax.ShapeDtypeStruct((oc, NPAD, 128), F32),
        scratch_shapes=[pltpu.VMEM((MBLK, m_out), F32)],
        compiler_params=pltpu.CompilerParams(
            dimension_semantics=("parallel", "arbitrary")),
    )(a1, w1, bias)


def _h2enc(a2, y2r2, batch3):
    def body(a_ref, r_ref, bt_ref, h_ref, e_ref, hs, cs):
        m = pl.program_id(0)

        @pl.when(m == 0)
        def _():
            hs[...] = jnp.zeros_like(hs)
            cs[...] = jnp.zeros_like(cs)

        h = jnp.maximum(a_ref[0] + a_ref[1] + r_ref[0], 0.0)
        h_ref[...] = h
        bt = bt_ref[0, 0]
        grp = lax.broadcasted_iota(jnp.int32, (G, MBLK), 0)
        onehot = (bt[None, :] == grp).astype(F32)
        hs[...] += jnp.dot(onehot, h, preferred_element_type=F32)
        cnt = jnp.sum(onehot, axis=1, keepdims=True)
        cs[...] += jnp.broadcast_to(cnt, (G, 128))

        @pl.when(m == NBLK - 1)
        def _():
            e_ref[...] = hs[...] / jnp.maximum(cs[...], 1.0)

    return pl.pallas_call(
        body,
        grid=(NBLK,),
        in_specs=[
            pl.BlockSpec((2, MBLK, 128), lambda m: (0, m, 0)),
            pl.BlockSpec((1, MBLK, 128), lambda m: (1, m, 0)),
            pl.BlockSpec((1, 1, MBLK), lambda m: (m, 0, 0)),
        ],
        out_specs=[
            pl.BlockSpec((MBLK, 128), lambda m: (m, 0)),
            pl.BlockSpec((G, 128), lambda m: (0, 0)),
        ],
        out_shape=[
            jax.ShapeDtypeStruct((NPAD, 128), F32),
            jax.ShapeDtypeStruct((G, 128), F32),
        ],
        scratch_shapes=[pltpu.VMEM((G, 128), F32), pltpu.VMEM((G, 128), F32)],
        compiler_params=pltpu.CompilerParams(
            dimension_semantics=("arbitrary",)),
    )(a2, y2r2, batch3)


def _final(a5, y5r5):
    def body(a_ref, r_ref, o_ref):
        o_ref[...] = a_ref[0] + a_ref[1] + r_ref[0]

    return pl.pallas_call(
        body,
        grid=(NBLK,),
        in_specs=[
            pl.BlockSpec((2, MBLK, 128), lambda m: (0, m, 0)),
            pl.BlockSpec((1, MBLK, 128), lambda m: (1, m, 0)),
        ],
        out_specs=pl.BlockSpec((MBLK, 128), lambda m: (m, 0)),
        out_shape=jax.ShapeDtypeStruct((NPAD, 128), F32),
    )(a5, y5r5)


def _padw(w, r, c):
    return jnp.pad(w, ((0, r - w.shape[0]), (0, c - w.shape[1])))


def _padb(b, m_out):
    return jnp.pad(b, (0, m_out - b.shape[0])).reshape(1, m_out)


def kernel(x, edge_index, edge_attr, batch,
           w0_rel, b0, w0_root, w1_rel, b1, w1_root, w2_rel, b2, w2_root,
           w3_rel, b3, w3_root, w4_rel, b4, w4_root, w5_rel, b5, w5_root):
    src = edge_index[0]
    dst = edge_index[1]
    pad_e = EPAD - E
    src_p = jnp.concatenate([src, jnp.zeros((pad_e,), jnp.int32)])
    dst_p = jnp.concatenate([dst, jnp.full((pad_e,), TRASH, jnp.int32)])
    src_all = (src_p[None, :]
               + (jnp.arange(HC, dtype=jnp.int32) * NPAD)[:, None]
               ).reshape(HC * EPAD // EB, EB)
    src_p = src_p.reshape(EPAD // EB, EB)
    dst_p = dst_p.reshape(EPAD // EB, EB)
    batch3 = jnp.concatenate(
        [batch, jnp.full((NPAD - N,), G, jnp.int32)]).reshape(NBLK, 1, MBLK)
    xp = jnp.pad(x, ((0, NPAD - N), (0, 0)))
    zrows = jnp.zeros((TPR, 128), F32)

    w0a = _padw(jnp.concatenate([w0_rel, w0_rel], axis=0), 256, HP)
    w0b = _padw(w0_root, 128, HP)
    bias0 = _padb(b0, HP)
    w1a = _padw(w1_rel, HP, HP)
    w1b = _padw(w1_root, HP, HP)
    bias1 = _padb(b1, HP)
    w2c = _padw(jnp.concatenate([w2_rel, w2_root], axis=1), HP, 256)
    bias2 = jnp.concatenate([jnp.zeros((128,), F32), b2]).reshape(1, 256)
    w3a = _padw(jnp.concatenate([w3_rel, w3_rel], axis=0), 256, HP)
    w3b = _padw(w3_root, 128, HP)
    bias3 = _padb(b3, HP)
    w4a = _padw(w4_rel, HP, HP)
    w4b = _padw(w4_root, HP, HP)
    bias4 = _padb(b4, HP)
    w5c = _padw(jnp.concatenate([w5_rel, w5_root], axis=1), HP, 256)
    bias5 = jnp.concatenate([jnp.zeros((128,), F32), b5]).reshape(1, 256)

    a0 = _seg128(xp, src_p, dst_p, zrows)
    h0 = _mm2(a0, w0a, xp.reshape(1, NPAD, 128), w0b, bias0, True)
    a1 = _segchunk(h0.reshape(HC * NPAD, 128), src_all, dst_p, zrows)
    h1 = _mm2(a1, w1a, h0, w1b, bias1, True)
    y2r2 = _mm1(h1, w2c, bias2, False)
    a2 = _seg128(y2r2.reshape(2 * NPAD, 128), src_p, dst_p, zrows)
    h2, enc = _h2enc(a2, y2r2, batch3)
    a3 = _seg128(h2, src_p, dst_p, zrows)
    h3 = _mm2(a3, w3a, h2.reshape(1, NPAD, 128), w3b, bias3, True)
    a4 = _segchunk(h3.reshape(HC * NPAD, 128), src_all, dst_p, zrows)
    h4 = _mm2(a4, w4a, h3, w4b, bias4, True)
    y5r5 = _mm1(h4, w5c, bias5, False)
    a5 = _seg128(y5r5.reshape(2 * NPAD, 128), src_p, dst_p, zrows)
    out = _final(a5, y5r5)
    return (out[:N], enc)


# speedup vs baseline: 3.2564x; 1.0502x over previous
import functools

import jax
import jax.numpy as jnp
from jax import lax
from jax.experimental import pallas as pl
from jax.experimental.pallas import tpu as pltpu
from jax.experimental.pallas import tpu_sc as plsc

N = 10000
E = 320000
G = 16
NPAD = 10240
TRASH = N
MBLK = 256
NBLK = NPAD // MBLK
EB = 128
EPAD = 327680
TPR = NPAD // 16
HP = 2048
HC = HP // 128
F32 = jnp.float32


def _seg_mesh():
    return plsc.VectorSubcoreMesh(core_axis_name="c", subcore_axis_name="s",
                                  num_cores=2, num_subcores=16)


NRING = 2
IB = 40


def _edge_pipeline(table_r, shared, src_hbm, src_base, dst_hbm, dst_base,
                   src_v, dst_v, rows, gsems, ssems, nb):
    def gather(i, b):
        pltpu.async_copy(table_r.at[src_v.at[i]], rows.at[b], gsems[b])

    def wait_gather(b):
        pltpu.make_async_copy(table_r.at[src_v.at[0]], rows.at[b],
                              gsems[b]).wait()

    def scatter(i, b):
        pltpu.async_copy(rows.at[b], shared.at[dst_v.at[i]], ssems[b],
                         add=True)

    def wait_scatter(b):
        pltpu.make_async_copy(rows.at[b], shared.at[dst_v.at[0]],
                              ssems[b]).wait()

    def block(j2, carry):
        pltpu.sync_copy(src_hbm.at[pl.ds(src_base + j2 * IB, IB)], src_v)
        pltpu.sync_copy(dst_hbm.at[pl.ds(dst_base + j2 * IB, IB)], dst_v)
        for b in range(NRING):
            gather(b, b)

        def body(i, c2):
            for b in range(NRING):
                wait_gather(b)
                scatter(i + b, b)
            for b in range(NRING):
                wait_scatter(b)

                @pl.when(i + NRING + b < IB)
                def _():
                    gather(i + NRING + b, b)

            return c2

        lax.fori_loop(0, IB // NRING, lambda i2, c2: body(i2 * NRING, c2), 0)
        return carry

    lax.fori_loop(0, nb // IB, block, 0)


def _seg128(table, src2d, dst2d, zrows):
    ept = EPAD // 32
    nb = ept // EB

    @functools.partial(
        pl.kernel,
        mesh=_seg_mesh(),
        out_type=jax.ShapeDtypeStruct((2 * NPAD, 128), F32),
        scratch_types=[
            pltpu.VMEM((IB, EB), jnp.int32),
            pltpu.VMEM((IB, EB), jnp.int32),
            pltpu.VMEM((NRING, EB, 128), F32),
            pltpu.VMEM_SHARED((NPAD, 128), F32),
        ] + [pltpu.SemaphoreType.DMA] * (2 * NRING),
    )
    def k(table_r, src_r, dst_r, z_r, out_r, src_v, dst_v, rows, shared,
          *sems):
        c = lax.axis_index("c")
        s = lax.axis_index("s")
        g = c * 16 + s
        pltpu.sync_copy(z_r, shared.at[pl.ds(s * TPR, TPR)])
        plsc.subcore_barrier()
        _edge_pipeline(table_r, shared, src_r, g * nb, dst_r, g * nb,
                       src_v, dst_v, rows, sems[:NRING], sems[NRING:], nb)
        plsc.subcore_barrier()
        pltpu.sync_copy(
            shared.at[pl.ds(s * TPR, TPR)],
            out_r.at[pl.ds(c * NPAD + s * TPR, TPR)],
        )

    return k(table, src2d, dst2d, zrows).reshape(2, NPAD, 128)


def _segchunk(table, src_all, dst2d, zrows):
    ept = EPAD // 16
    nb = ept // EB
    nbt = EPAD // EB

    @functools.partial(
        pl.kernel,
        mesh=_seg_mesh(),
        out_type=jax.ShapeDtypeStruct((HC * NPAD, 128), F32),
        scratch_types=[
            pltpu.VMEM((IB, EB), jnp.int32),
            pltpu.VMEM((IB, EB), jnp.int32),
            pltpu.VMEM((NRING, EB, 128), F32),
            pltpu.VMEM_SHARED((NPAD, 128), F32),
        ] + [pltpu.SemaphoreType.DMA] * (2 * NRING),
    )
    def k(table_r, src_r, dst_r, z_r, out_r, src_v, dst_v, rows, shared,
          *sems):
        c = lax.axis_index("c")
        s = lax.axis_index("s")

        def chunk_body(j, carry):
            ch = c * 8 + j
            pltpu.sync_copy(z_r, shared.at[pl.ds(s * TPR, TPR)])
            plsc.subcore_barrier()
            _edge_pipeline(table_r, shared, src_r, ch * nbt + s * nb,
                           dst_r, s * nb, src_v, dst_v, rows,
                           sems[:NRING], sems[NRING:], nb)
            plsc.subcore_barrier()
            pltpu.sync_copy(
                shared.at[pl.ds(s * TPR, TPR)],
                out_r.at[pl.ds(ch * NPAD + s * TPR, TPR)],
            )
            return carry

        lax.fori_loop(0, 8, chunk_body, 0)

    return k(table, src_all, dst2d, zrows).reshape(HC, NPAD, 128)


def _mm2(a1, w1, a2, w2, bias, relu):
    k1 = a1.shape[0]
    k2 = a2.shape[0]
    nk = k1 + k2
    m_out = w1.shape[1]
    oc = m_out // 128

    def body(a1_ref, w1_ref, a2_ref, w2_ref, b_ref, o_ref, acc):
        ki = pl.program_id(1)

        @pl.when(ki == 0)
        def _():
            acc[...] = jnp.zeros_like(acc)

        @pl.when(ki < k1)
        def _():
            acc[...] += jnp.dot(a1_ref[0], w1_ref[...],
                                preferred_element_type=F32)

        @pl.when(ki >= k1)
        def _():
            acc[...] += jnp.dot(a2_ref[0], w2_ref[...],
                                preferred_element_type=F32)

        @pl.when(ki == nk - 1)
        def _():
            z = acc[...] + b_ref[...]
            if relu:
                z = jnp.maximum(z, 0.0)
            for ci in range(oc):
                o_ref[ci] = z[:, ci * 128:(ci + 1) * 128]

    return pl.pallas_call(
        body,
        grid=(NBLK, nk),
        in_specs=[
            pl.BlockSpec((1, MBLK, 128), lambda m, k: (jnp.minimum(k, k1 - 1), m, 0)),
            pl.BlockSpec((128, m_out), lambda m, k: (jnp.minimum(k, k1 - 1), 0)),
            pl.BlockSpec((1, MBLK, 128), lambda m, k: (jnp.maximum(k - k1, 0), m, 0)),
            pl.BlockSpec((128, m_out), lambda m, k: (jnp.maximum(k - k1, 0), 0)),
            pl.BlockSpec((1, m_out), lambda m, k: (0, 0)),
        ],
        out_specs=pl.BlockSpec((oc, MBLK, 128), lambda m, k: (0, m, 0)),
        out_shape=jax.ShapeDtypeStruct((oc, NPAD, 128), F32),
        scratch_shapes=[pltpu.VMEM((MBLK, m_out), F32)],
        compiler_params=pltpu.CompilerParams(
            dimension_semantics=("parallel", "arbitrary")),
    )(a1, w1, a2, w2, bias)


def _mm1(a1, w1, bias, relu):
    nk = a1.shape[0]
    m_out = w1.shape[1]
    oc = m_out // 128

    def body(a1_ref, w1_ref, b_ref, o_ref, acc):
        ki = pl.program_id(1)

        @pl.when(ki == 0)
        def _():
            acc[...] = jnp.zeros_like(acc)

        acc[...] += jnp.dot(a1_ref[0], w1_ref[...], preferred_element_type=F32)

        @pl.when(ki == nk - 1)
        def _():
            z = acc[...] + b_ref[...]
            if relu:
                z = jnp.maximum(z, 0.0)
            for ci in range(oc):
                o_ref[ci] = z[:, ci * 128:(ci + 1) * 128]

    return pl.pallas_call(
        body,
        grid=(NBLK, nk),
        in_specs=[
            pl.BlockSpec((1, MBLK, 128), lambda m, k: (k, m, 0)),
            pl.BlockSpec((128, m_out), lambda m, k: (k, 0)),
            pl.BlockSpec((1, m_out), lambda m, k: (0, 0)),
        ],
        out_specs=pl.BlockSpec((oc, MBLK, 128), lambda m, k: (0, m, 0)),
        out_shape=jax.ShapeDtypeStruct((oc, NPAD, 128), F32),
        scratch_shapes=[pltpu.VMEM((MBLK, m_out), F32)],
        compiler_params=pltpu.CompilerParams(
            dimension_semantics=("parallel", "arbitrary")),
    )(a1, w1, bias)


def _mm_add(a1, w1, r, relu):
    nk = a1.shape[0]
    m_out = w1.shape[1]
    oc = m_out // 128

    def body(a1_ref, w1_ref, r_ref, o_ref, acc):
        ki = pl.program_id(1)

        @pl.when(ki == 0)
        def _():
            acc[...] = jnp.zeros_like(acc)

        acc[...] += jnp.dot(a1_ref[0], w1_ref[...], preferred_element_type=F32)

        @pl.when(ki == nk - 1)
        def _():
            for ci in range(oc):
                z = acc[:, ci * 128:(ci + 1) * 128] + r_ref[ci]
                if relu:
                    z = jnp.maximum(z, 0.0)
                o_ref[ci] = z

    return pl.pallas_call(
        body,
        grid=(NBLK, nk),
        in_specs=[
            pl.BlockSpec((1, MBLK, 128), lambda m, k: (k, m, 0)),
            pl.BlockSpec((128, m_out), lambda m, k: (k, 0)),
            pl.BlockSpec((oc, MBLK, 128), lambda m, k: (0, m, 0)),
        ],
        out_specs=pl.BlockSpec((oc, MBLK, 128), lambda m, k: (0, m, 0)),
        out_shape=jax.ShapeDtypeStruct((oc, NPAD, 128), F32),
        scratch_shapes=[pltpu.VMEM((MBLK, m_out), F32)],
        compiler_params=pltpu.CompilerParams(
            dimension_semantics=("parallel", "arbitrary")),
    )(a1, w1, r)


def _h2enc(a2, y2r2, batch3):
    def body(a_ref, r_ref, bt_ref, h_ref, e_ref, hs, cs):
        m = pl.program_id(0)

        @pl.when(m == 0)
        def _():
            hs[...] = jnp.zeros_like(hs)
            cs[...] = jnp.zeros_like(cs)

        h = jnp.maximum(a_ref[0] + a_ref[1] + r_ref[0], 0.0)
        h_ref[...] = h
        bt = bt_ref[0, 0]
        grp = lax.broadcasted_iota(jnp.int32, (G, MBLK), 0)
        onehot = (bt[None, :] == grp).astype(F32)
        hs[...] += jnp.dot(onehot, h, preferred_element_type=F32)
        cnt = jnp.sum(onehot, axis=1, keepdims=True)
        cs[...] += jnp.broadcast_to(cnt, (G, 128))

        @pl.when(m == NBLK - 1)
        def _():
            e_ref[...] = hs[...] / jnp.maximum(cs[...], 1.0)

    return pl.pallas_call(
        body,
        grid=(NBLK,),
        in_specs=[
            pl.BlockSpec((2, MBLK, 128), lambda m: (0, m, 0)),
            pl.BlockSpec((1, MBLK, 128), lambda m: (0, m, 0)),
            pl.BlockSpec((1, 1, MBLK), lambda m: (m, 0, 0)),
        ],
        out_specs=[
            pl.BlockSpec((MBLK, 128), lambda m: (m, 0)),
            pl.BlockSpec((G, 128), lambda m: (0, 0)),
        ],
        out_shape=[
            jax.ShapeDtypeStruct((NPAD, 128), F32),
            jax.ShapeDtypeStruct((G, 128), F32),
        ],
        scratch_shapes=[pltpu.VMEM((G, 128), F32), pltpu.VMEM((G, 128), F32)],
        compiler_params=pltpu.CompilerParams(
            dimension_semantics=("arbitrary",)),
    )(a2, y2r2, batch3)


def _final(a5, y5r5):
    def body(a_ref, r_ref, o_ref):
        o_ref[...] = a_ref[0] + a_ref[1] + r_ref[0]

    return pl.pallas_call(
        body,
        grid=(NBLK,),
        in_specs=[
            pl.BlockSpec((2, MBLK, 128), lambda m: (0, m, 0)),
            pl.BlockSpec((1, MBLK, 128), lambda m: (0, m, 0)),
        ],
        out_specs=pl.BlockSpec((MBLK, 128), lambda m: (m, 0)),
        out_shape=jax.ShapeDtypeStruct((NPAD, 128), F32),
    )(a5, y5r5)


def _padw(w, r, c):
    return jnp.pad(w, ((0, r - w.shape[0]), (0, c - w.shape[1])))


def _padb(b, m_out):
    return jnp.pad(b, (0, m_out - b.shape[0])).reshape(1, m_out)


def kernel(x, edge_index, edge_attr, batch,
           w0_rel, b0, w0_root, w1_rel, b1, w1_root, w2_rel, b2, w2_root,
           w3_rel, b3, w3_root, w4_rel, b4, w4_root, w5_rel, b5, w5_root):
    src = edge_index[0]
    dst = edge_index[1]
    pad_e = EPAD - E
    src_p = jnp.concatenate([src, jnp.zeros((pad_e,), jnp.int32)])
    dst_p = jnp.concatenate([dst, jnp.full((pad_e,), TRASH, jnp.int32)])
    src_all = (src_p[None, :]
               + (jnp.arange(HC, dtype=jnp.int32) * NPAD)[:, None]
               ).reshape(HC * EPAD // EB, EB)
    src_p = src_p.reshape(EPAD // EB, EB)
    dst_p = dst_p.reshape(EPAD // EB, EB)
    batch3 = jnp.concatenate(
        [batch, jnp.full((NPAD - N,), G, jnp.int32)]).reshape(NBLK, 1, MBLK)
    xp = jnp.pad(x, ((0, NPAD - N), (0, 0)))
    zrows = jnp.zeros((TPR, 128), F32)

    w0a = _padw(jnp.concatenate([w0_rel, w0_rel], axis=0), 256, HP)
    w0b = _padw(w0_root, 128, HP)
    bias0 = _padb(b0, HP)
    w1a = _padw(w1_rel, HP, HP)
    w1b = _padw(w1_root, HP, HP)
    bias1 = _padb(b1, HP)
    w2a = _padw(w2_rel, HP, 128)
    w2b = _padw(w2_root, HP, 128)
    bias2 = b2.reshape(1, 128)
    zb128 = jnp.zeros((1, 128), F32)
    w3a = _padw(jnp.concatenate([w3_rel, w3_rel], axis=0), 256, HP)
    w3b = _padw(w3_root, 128, HP)
    bias3 = _padb(b3, HP)
    w4a = _padw(w4_rel, HP, HP)
    w4b = _padw(w4_root, HP, HP)
    bias4 = _padb(b4, HP)
    w5a = _padw(w5_rel, HP, 128)
    w5b = _padw(w5_root, HP, 128)
    bias5 = b5.reshape(1, 128)

    xc = xp.reshape(1, NPAD, 128)
    a0 = _seg128(xp, src_p, dst_p, zrows)
    r0 = _mm1(xc, w0b, bias0, False)
    h0 = _mm_add(a0, w0a, r0, True)
    a1 = _segchunk(h0.reshape(HC * NPAD, 128), src_all, dst_p, zrows)
    r1 = _mm1(h0, w1b, bias1, False)
    h1 = _mm_add(a1, w1a, r1, True)
    y2 = _mm1(h1, w2a, zb128, False)
    a2 = _seg128(y2.reshape(NPAD, 128), src_p, dst_p, zrows)
    r2 = _mm1(h1, w2b, bias2, False)
    h2, enc = _h2enc(a2, r2, batch3)
    a3 = _seg128(h2, src_p, dst_p, zrows)
    r3 = _mm1(h2.reshape(1, NPAD, 128), w3b, bias3, False)
    h3 = _mm_add(a3, w3a, r3, True)
    a4 = _segchunk(h3.reshape(HC * NPAD, 128), src_all, dst_p, zrows)
    r4 = _mm1(h3, w4b, bias4, False)
    h4 = _mm_add(a4, w4a, r4, True)
    y5 = _mm1(h4, w5a, zb128, False)
    a5 = _seg128(y5.reshape(NPAD, 128), src_p, dst_p, zrows)
    r5 = _mm1(h4, w5b, bias5, False)
    out = _final(a5, r5)
    return (out[:N], enc)


# speedup vs baseline: 3.3135x; 1.0175x over previous
import functools

import jax
import jax.numpy as jnp
from jax import lax
from jax.experimental import pallas as pl
from jax.experimental.pallas import tpu as pltpu
from jax.experimental.pallas import tpu_sc as plsc

N = 10000
E = 320000
G = 16
NPAD = 10240
TRASH = N
MBLK = 256
NBLK = NPAD // MBLK
EB = 128
EPAD = 327680
SROWS = 10240
TPR = SROWS // 16
HP = 2048
HC = HP // 128
F32 = jnp.float32


def _seg_mesh():
    return plsc.VectorSubcoreMesh(core_axis_name="c", subcore_axis_name="s",
                                  num_cores=2, num_subcores=16)


NRING = 2
IB = 40


def _edge_pipeline(table_r, shared, src_hbm, src_base, dst_hbm, dst_base,
                   src_v, dst_v, rows, gsems, ssems, nb):
    def gather(i, b):
        pltpu.async_copy(table_r.at[src_v.at[i]], rows.at[b], gsems[b])

    def wait_gather(b):
        pltpu.make_async_copy(table_r.at[src_v.at[0]], rows.at[b],
                              gsems[b]).wait()

    def scatter(i, b):
        pltpu.async_copy(rows.at[b], shared.at[dst_v.at[i]], ssems[b],
                         add=True)

    def wait_scatter(b):
        pltpu.make_async_copy(rows.at[b], shared.at[dst_v.at[0]],
                              ssems[b]).wait()

    def block(j2, carry):
        pltpu.sync_copy(src_hbm.at[pl.ds(src_base + j2 * IB, IB)], src_v)
        pltpu.sync_copy(dst_hbm.at[pl.ds(dst_base + j2 * IB, IB)], dst_v)
        for b in range(NRING):
            gather(b, b)

        def body(i, c2):
            for b in range(NRING):
                wait_gather(b)
                scatter(i + b, b)
            for b in range(NRING):
                wait_scatter(b)

                @pl.when(i + NRING + b < IB)
                def _():
                    gather(i + NRING + b, b)

            return c2

        lax.fori_loop(0, IB // NRING, lambda i2, c2: body(i2 * NRING, c2), 0)
        return carry

    lax.fori_loop(0, nb // IB, block, 0)


def _seg128(table, src2d, dst2d, zrows):
    ept = EPAD // 32
    nb = ept // EB

    @functools.partial(
        pl.kernel,
        mesh=_seg_mesh(),
        out_type=jax.ShapeDtypeStruct((2 * NPAD, 128), F32),
        scratch_types=[
            pltpu.VMEM((IB, EB), jnp.int32),
            pltpu.VMEM((IB, EB), jnp.int32),
            pltpu.VMEM((NRING, EB, 128), F32),
            pltpu.VMEM_SHARED((SROWS, 128), F32),
        ] + [pltpu.SemaphoreType.DMA] * (2 * NRING),
    )
    def k(table_r, src_r, dst_r, z_r, out_r, src_v, dst_v, rows, shared,
          *sems):
        c = lax.axis_index("c")
        s = lax.axis_index("s")
        g = c * 16 + s
        pltpu.sync_copy(z_r, shared.at[pl.ds(s * TPR, TPR)])
        plsc.subcore_barrier()
        _edge_pipeline(table_r, shared, src_r, g * nb, dst_r, g * nb,
                       src_v, dst_v, rows, sems[:NRING], sems[NRING:], nb)
        plsc.subcore_barrier()
        pltpu.sync_copy(
            shared.at[pl.ds(s * TPR, TPR)],
            out_r.at[pl.ds(c * NPAD + s * TPR, TPR)],
        )

    return k(table, src2d, dst2d, zrows).reshape(2, NPAD, 128)


def _segchunk(table, src_all, dst2d, zrows, base, nch):
    ept = EPAD // 16
    nb = ept // EB
    nbt = EPAD // EB
    cpc = nch // 2

    @functools.partial(
        pl.kernel,
        mesh=_seg_mesh(),
        out_type=jax.ShapeDtypeStruct((nch * NPAD, 128), F32),
        scratch_types=[
            pltpu.VMEM((IB, EB), jnp.int32),
            pltpu.VMEM((IB, EB), jnp.int32),
            pltpu.VMEM((NRING, EB, 128), F32),
            pltpu.VMEM_SHARED((SROWS, 128), F32),
        ] + [pltpu.SemaphoreType.DMA] * (2 * NRING),
    )
    def k(table_r, src_r, dst_r, z_r, out_r, src_v, dst_v, rows, shared,
          *sems):
        c = lax.axis_index("c")
        s = lax.axis_index("s")

        def chunk_body(j, carry):
            ch = base + c * cpc + j
            och = c * cpc + j
            pltpu.sync_copy(z_r, shared.at[pl.ds(s * TPR, TPR)])
            plsc.subcore_barrier()
            _edge_pipeline(table_r, shared, src_r, ch * nbt + s * nb,
                           dst_r, s * nb, src_v, dst_v, rows,
                           sems[:NRING], sems[NRING:], nb)
            plsc.subcore_barrier()
            pltpu.sync_copy(
                shared.at[pl.ds(s * TPR, TPR)],
                out_r.at[pl.ds(och * NPAD + s * TPR, TPR)],
            )
            return carry

        lax.fori_loop(0, cpc, chunk_body, 0)

    return k(table, src_all, dst2d, zrows).reshape(nch, NPAD, 128)


def _mm2(a1, w1, a2, w2, bias, relu):
    k1 = a1.shape[0]
    k2 = a2.shape[0]
    nk = k1 + k2
    m_out = w1.shape[1]
    oc = m_out // 128

    def body(a1_ref, w1_ref, a2_ref, w2_ref, b_ref, o_ref, acc):
        ki = pl.program_id(1)

        @pl.when(ki == 0)
        def _():
            acc[...] = jnp.zeros_like(acc)

        @pl.when(ki < k1)
        def _():
            acc[...] += jnp.dot(a1_ref[0], w1_ref[...],
                                preferred_element_type=F32)

        @pl.when(ki >= k1)
        def _():
            acc[...] += jnp.dot(a2_ref[0], w2_ref[...],
                                preferred_element_type=F32)

        @pl.when(ki == nk - 1)
        def _():
            z = acc[...] + b_ref[...]
            if relu:
                z = jnp.maximum(z, 0.0)
            for ci in range(oc):
                o_ref[ci] = z[:, ci * 128:(ci + 1) * 128]

    return pl.pallas_call(
        body,
        grid=(NBLK, nk),
        in_specs=[
            pl.BlockSpec((1, MBLK, 128), lambda m, k: (jnp.minimum(k, k1 - 1), m, 0)),
            pl.BlockSpec((128, m_out), lambda m, k: (jnp.minimum(k, k1 - 1), 0)),
            pl.BlockSpec((1, MBLK, 128), lambda m, k: (jnp.maximum(k - k1, 0), m, 0)),
            pl.BlockSpec((128, m_out), lambda m, k: (jnp.maximum(k - k1, 0), 0)),
            pl.BlockSpec((1, m_out), lambda m, k: (0, 0)),
        ],
        out_specs=pl.BlockSpec((oc, MBLK, 128), lambda m, k: (0, m, 0)),
        out_shape=jax.ShapeDtypeStruct((oc, NPAD, 128), F32),
        scratch_shapes=[pltpu.VMEM((MBLK, m_out), F32)],
        compiler_params=pltpu.CompilerParams(
            dimension_semantics=("parallel", "arbitrary")),
    )(a1, w1, a2, w2, bias)


def _mm1(a1, w1, bias, relu):
    nk = a1.shape[0]
    m_out = w1.shape[1]
    oc = m_out // 128

    def body(a1_ref, w1_ref, b_ref, o_ref, acc):
        ki = pl.program_id(1)

        @pl.when(ki == 0)
        def _():
            acc[...] = jnp.zeros_like(acc)

        acc[...] += jnp.dot(a1_ref[0], w1_ref[...], preferred_element_type=F32)

        @pl.when(ki == nk - 1)
        def _():
            z = acc[...] + b_ref[...]
            if relu:
                z = jnp.maximum(z, 0.0)
            for ci in range(oc):
                o_ref[ci] = z[:, ci * 128:(ci + 1) * 128]

    return pl.pallas_call(
        body,
        grid=(NBLK, nk),
        in_specs=[
            pl.BlockSpec((1, MBLK, 128), lambda m, k: (k, m, 0)),
            pl.BlockSpec((128, m_out), lambda m, k: (k, 0)),
            pl.BlockSpec((1, m_out), lambda m, k: (0, 0)),
        ],
        out_specs=pl.BlockSpec((oc, MBLK, 128), lambda m, k: (0, m, 0)),
        out_shape=jax.ShapeDtypeStruct((oc, NPAD, 128), F32),
        scratch_shapes=[pltpu.VMEM((MBLK, m_out), F32)],
        compiler_params=pltpu.CompilerParams(
            dimension_semantics=("parallel", "arbitrary")),
    )(a1, w1, bias)


def _mm_add(a1, w1, rs, relu):
    nk = a1.shape[0]
    m_out = w1.shape[1]
    oc = m_out // 128
    nr = len(rs)

    def body(a1_ref, w1_ref, *rest):
        r_refs = rest[:nr]
        o_ref = rest[nr]
        acc = rest[nr + 1]
        ki = pl.program_id(1)

        @pl.when(ki == 0)
        def _():
            acc[...] = jnp.zeros_like(acc)

        acc[...] += jnp.dot(a1_ref[0], w1_ref[...], preferred_element_type=F32)

        @pl.when(ki == nk - 1)
        def _():
            for ci in range(oc):
                z = acc[:, ci * 128:(ci + 1) * 128]
                for r_ref in r_refs:
                    z = z + r_ref[ci]
                if relu:
                    z = jnp.maximum(z, 0.0)
                o_ref[ci] = z

    return pl.pallas_call(
        body,
        grid=(NBLK, nk),
        in_specs=[
            pl.BlockSpec((1, MBLK, 128), lambda m, k: (k, m, 0)),
            pl.BlockSpec((128, m_out), lambda m, k: (k, 0)),
        ] + [pl.BlockSpec((oc, MBLK, 128), lambda m, k: (0, m, 0))] * nr,
        out_specs=pl.BlockSpec((oc, MBLK, 128), lambda m, k: (0, m, 0)),
        out_shape=jax.ShapeDtypeStruct((oc, NPAD, 128), F32),
        scratch_shapes=[pltpu.VMEM((MBLK, m_out), F32)],
        compiler_params=pltpu.CompilerParams(
            dimension_semantics=("parallel", "arbitrary")),
    )(a1, w1, *rs)


def _h2enc(a2, y2r2, batch3):
    def body(a_ref, r_ref, bt_ref, h_ref, e_ref, hs, cs):
        m = pl.program_id(0)

        @pl.when(m == 0)
        def _():
            hs[...] = jnp.zeros_like(hs)
            cs[...] = jnp.zeros_like(cs)

        h = jnp.maximum(a_ref[0] + a_ref[1] + r_ref[0], 0.0)
        h_ref[...] = h
        bt = bt_ref[0, 0]
        grp = lax.broadcasted_iota(jnp.int32, (G, MBLK), 0)
        onehot = (bt[None, :] == grp).astype(F32)
        hs[...] += jnp.dot(onehot, h, preferred_element_type=F32)
        cnt = jnp.sum(onehot, axis=1, keepdims=True)
        cs[...] += jnp.broadcast_to(cnt, (G, 128))

        @pl.when(m == NBLK - 1)
        def _():
            e_ref[...] = hs[...] / jnp.maximum(cs[...], 1.0)

    return pl.pallas_call(
        body,
        grid=(NBLK,),
        in_specs=[
            pl.BlockSpec((2, MBLK, 128), lambda m: (0, m, 0)),
            pl.BlockSpec((1, MBLK, 128), lambda m: (0, m, 0)),
            pl.BlockSpec((1, 1, MBLK), lambda m: (m, 0, 0)),
        ],
        out_specs=[
            pl.BlockSpec((MBLK, 128), lambda m: (m, 0)),
            pl.BlockSpec((G, 128), lambda m: (0, 0)),
        ],
        out_shape=[
            jax.ShapeDtypeStruct((NPAD, 128), F32),
            jax.ShapeDtypeStruct((G, 128), F32),
        ],
        scratch_shapes=[pltpu.VMEM((G, 128), F32), pltpu.VMEM((G, 128), F32)],
        compiler_params=pltpu.CompilerParams(
            dimension_semantics=("arbitrary",)),
    )(a2, y2r2, batch3)


def _final(a5, y5r5):
    def body(a_ref, r_ref, o_ref):
        o_ref[...] = a_ref[0] + a_ref[1] + r_ref[0]

    return pl.pallas_call(
        body,
        grid=(NBLK,),
        in_specs=[
            pl.BlockSpec((2, MBLK, 128), lambda m: (0, m, 0)),
            pl.BlockSpec((1, MBLK, 128), lambda m: (0, m, 0)),
        ],
        out_specs=pl.BlockSpec((MBLK, 128), lambda m: (m, 0)),
        out_shape=jax.ShapeDtypeStruct((NPAD, 128), F32),
    )(a5, y5r5)


def _padw(w, r, c):
    return jnp.pad(w, ((0, r - w.shape[0]), (0, c - w.shape[1])))


def _padb(b, m_out):
    return jnp.pad(b, (0, m_out - b.shape[0])).reshape(1, m_out)


def kernel(x, edge_index, edge_attr, batch,
           w0_rel, b0, w0_root, w1_rel, b1, w1_root, w2_rel, b2, w2_root,
           w3_rel, b3, w3_root, w4_rel, b4, w4_root, w5_rel, b5, w5_root):
    src = edge_index[0]
    dst = edge_index[1]
    pad_e = EPAD - E
    src_p = jnp.concatenate([src, jnp.zeros((pad_e,), jnp.int32)])
    dst_p = jnp.concatenate([dst, jnp.full((pad_e,), TRASH, jnp.int32)])
    src_all = (src_p[None, :]
               + (jnp.arange(HC, dtype=jnp.int32) * NPAD)[:, None]
               ).reshape(HC * EPAD // EB, EB)
    src_p = src_p.reshape(EPAD // EB, EB)
    dst_p = dst_p.reshape(EPAD // EB, EB)
    batch3 = jnp.concatenate(
        [batch, jnp.full((NPAD - N,), G, jnp.int32)]).reshape(NBLK, 1, MBLK)
    xp = jnp.pad(x, ((0, NPAD - N), (0, 0)))
    zrows = jnp.zeros((TPR, 128), F32)

    w0a = _padw(jnp.concatenate([w0_rel, w0_rel], axis=0), 256, HP)
    w0b = _padw(w0_root, 128, HP)
    bias0 = _padb(b0, HP)
    w1a = _padw(w1_rel, HP, HP)
    w1b = _padw(w1_root, HP, HP)
    bias1 = _padb(b1, HP)
    w2a = _padw(w2_rel, HP, 128)
    w2b = _padw(w2_root, HP, 128)
    bias2 = b2.reshape(1, 128)
    zb128 = jnp.zeros((1, 128), F32)
    w3a = _padw(jnp.concatenate([w3_rel, w3_rel], axis=0), 256, HP)
    w3b = _padw(w3_root, 128, HP)
    bias3 = _padb(b3, HP)
    w4a = _padw(w4_rel, HP, HP)
    w4b = _padw(w4_root, HP, HP)
    bias4 = _padb(b4, HP)
    w5a = _padw(w5_rel, HP, 128)
    w5b = _padw(w5_root, HP, 128)
    bias5 = b5.reshape(1, 128)

    xc = xp.reshape(1, NPAD, 128)
    zbH = jnp.zeros((1, HP), F32)
    w1a_top, w1a_bot = w1a[:HP // 2], w1a[HP // 2:]
    w4a_top, w4a_bot = w4a[:HP // 2], w4a[HP // 2:]

    a0 = _seg128(xp, src_p, dst_p, zrows)
    r0 = _mm1(xc, w0b, bias0, False)
    h0 = _mm_add(a0, w0a, [r0], True)
    h0f = h0.reshape(HC * NPAD, 128)
    a1a = _segchunk(h0f, src_all, dst_p, zrows, 0, 8)
    r1 = _mm1(h0, w1b, bias1, False)
    t1 = _mm1(a1a, w1a_top, zbH, False)
    a1b = _segchunk(h0f, src_all, dst_p, zrows, 8, 8)
    h1 = _mm_add(a1b, w1a_bot, [t1, r1], True)
    y2 = _mm1(h1, w2a, zb128, False)
    a2 = _seg128(y2.reshape(NPAD, 128), src_p, dst_p, zrows)
    r2 = _mm1(h1, w2b, bias2, False)
    h2, enc = _h2enc(a2, r2, batch3)
    a3 = _seg128(h2, src_p, dst_p, zrows)
    r3 = _mm1(h2.reshape(1, NPAD, 128), w3b, bias3, False)
    h3 = _mm_add(a3, w3a, [r3], True)
    h3f = h3.reshape(HC * NPAD, 128)
    a4a = _segchunk(h3f, src_all, dst_p, zrows, 0, 8)
    r4 = _mm1(h3, w4b, bias4, False)
    t4 = _mm1(a4a, w4a_top, zbH, False)
    a4b = _segchunk(h3f, src_all, dst_p, zrows, 8, 8)
    h4 = _mm_add(a4b, w4a_bot, [t4, r4], True)
    y5 = _mm1(h4, w5a, zb128, False)
    a5 = _seg128(y5.reshape(NPAD, 128), src_p, dst_p, zrows)
    r5 = _mm1(h4, w5b, bias5, False)
    out = _final(a5, r5)
    return (out[:N], enc)


# speedup vs baseline: 3.4514x; 1.0416x over previous
import functools

import jax
import jax.numpy as jnp
from jax import lax
from jax.experimental import pallas as pl
from jax.experimental.pallas import tpu as pltpu
from jax.experimental.pallas import tpu_sc as plsc

N = 10000
E = 320000
G = 16
NPAD = 10240
TRASH = N
MBLK = 256
NBLK = NPAD // MBLK
EB = 128
EPAD = 327680
SROWS = 10240
TPR = SROWS // 16
HP = 2048
HC = HP // 128
F32 = jnp.float32


def _seg_mesh():
    return plsc.VectorSubcoreMesh(core_axis_name="c", subcore_axis_name="s",
                                  num_cores=2, num_subcores=16)


NRING = 2
IB = 40


def _edge_pipeline(table_r, shared, src_hbm, src_base, dst_hbm, dst_base,
                   src_v, dst_v, rows, gsems, ssems, nb):
    def gather(i, b):
        pltpu.async_copy(table_r.at[src_v.at[i]], rows.at[b], gsems[b])

    def wait_gather(b):
        pltpu.make_async_copy(table_r.at[src_v.at[0]], rows.at[b],
                              gsems[b]).wait()

    def scatter(i, b):
        pltpu.async_copy(rows.at[b], shared.at[dst_v.at[i]], ssems[b],
                         add=True)

    def wait_scatter(b):
        pltpu.make_async_copy(rows.at[b], shared.at[dst_v.at[0]],
                              ssems[b]).wait()

    def block(j2, carry):
        pltpu.sync_copy(src_hbm.at[pl.ds(src_base + j2 * IB, IB)], src_v)
        pltpu.sync_copy(dst_hbm.at[pl.ds(dst_base + j2 * IB, IB)], dst_v)
        for b in range(NRING):
            gather(b, b)

        def body(i, c2):
            for b in range(NRING):
                wait_gather(b)
                scatter(i + b, b)
            for b in range(NRING):
                wait_scatter(b)

                @pl.when(i + NRING + b < IB)
                def _():
                    gather(i + NRING + b, b)

            return c2

        lax.fori_loop(0, IB // NRING, lambda i2, c2: body(i2 * NRING, c2), 0)
        return carry

    lax.fori_loop(0, nb // IB, block, 0)


def _seg128(table, src2d, dst2d, zrows):
    ept = EPAD // 32
    nb = ept // EB

    @functools.partial(
        pl.kernel,
        mesh=_seg_mesh(),
        out_type=jax.ShapeDtypeStruct((2 * NPAD, 128), F32),
        scratch_types=[
            pltpu.VMEM((IB, EB), jnp.int32),
            pltpu.VMEM((IB, EB), jnp.int32),
            pltpu.VMEM((NRING, EB, 128), F32),
            pltpu.VMEM_SHARED((SROWS, 128), F32),
        ] + [pltpu.SemaphoreType.DMA] * (2 * NRING),
    )
    def k(table_r, src_r, dst_r, z_r, out_r, src_v, dst_v, rows, shared,
          *sems):
        c = lax.axis_index("c")
        s = lax.axis_index("s")
        g = c * 16 + s
        pltpu.sync_copy(z_r, shared.at[pl.ds(s * TPR, TPR)])
        plsc.subcore_barrier()
        _edge_pipeline(table_r, shared, src_r, g * nb, dst_r, g * nb,
                       src_v, dst_v, rows, sems[:NRING], sems[NRING:], nb)
        plsc.subcore_barrier()
        pltpu.sync_copy(
            shared.at[pl.ds(s * TPR, TPR)],
            out_r.at[pl.ds(c * NPAD + s * TPR, TPR)],
        )

    return k(table, src2d, dst2d, zrows).reshape(2, NPAD, 128)


def _segchunk(table, src_all, dst2d, zrows, base, nch):
    ept = EPAD // 16
    nb = ept // EB
    nbt = EPAD // EB
    cpc = nch // 2

    @functools.partial(
        pl.kernel,
        mesh=_seg_mesh(),
        out_type=jax.ShapeDtypeStruct((nch * NPAD, 128), F32),
        scratch_types=[
            pltpu.VMEM((IB, EB), jnp.int32),
            pltpu.VMEM((IB, EB), jnp.int32),
            pltpu.VMEM((NRING, EB, 128), F32),
            pltpu.VMEM_SHARED((SROWS, 128), F32),
        ] + [pltpu.SemaphoreType.DMA] * (2 * NRING),
    )
    def k(table_r, src_r, dst_r, z_r, out_r, src_v, dst_v, rows, shared,
          *sems):
        c = lax.axis_index("c")
        s = lax.axis_index("s")

        def chunk_body(j, carry):
            ch = base + c * cpc + j
            och = c * cpc + j
            pltpu.sync_copy(z_r, shared.at[pl.ds(s * TPR, TPR)])
            plsc.subcore_barrier()
            _edge_pipeline(table_r, shared, src_r, ch * nbt + s * nb,
                           dst_r, s * nb, src_v, dst_v, rows,
                           sems[:NRING], sems[NRING:], nb)
            plsc.subcore_barrier()
            pltpu.sync_copy(
                shared.at[pl.ds(s * TPR, TPR)],
                out_r.at[pl.ds(och * NPAD + s * TPR, TPR)],
            )
            return carry

        lax.fori_loop(0, cpc, chunk_body, 0)

    return k(table, src_all, dst2d, zrows).reshape(nch, NPAD, 128)


def _mm1(a1, w1, bias, relu):
    nck = a1.shape[0]
    kc = 2 if nck % 2 == 0 else 1
    nk = nck // kc
    m_out = w1.shape[1]
    oc = m_out // 128

    def body(a1_ref, w1_ref, b_ref, o_ref, acc):
        ki = pl.program_id(1)

        @pl.when(ki == 0)
        def _():
            acc[...] = jnp.zeros_like(acc)

        a = jnp.concatenate([a1_ref[i] for i in range(kc)], axis=1) \
            if kc > 1 else a1_ref[0]
        acc[...] += jnp.dot(a, w1_ref[...], preferred_element_type=F32)

        @pl.when(ki == nk - 1)
        def _():
            z = acc[...] + b_ref[...]
            if relu:
                z = jnp.maximum(z, 0.0)
            for ci in range(oc):
                o_ref[ci] = z[:, ci * 128:(ci + 1) * 128]

    return pl.pallas_call(
        body,
        grid=(NBLK, nk),
        in_specs=[
            pl.BlockSpec((kc, MBLK, 128), lambda m, k: (k, m, 0)),
            pl.BlockSpec((kc * 128, m_out), lambda m, k: (k, 0)),
            pl.BlockSpec((1, m_out), lambda m, k: (0, 0)),
        ],
        out_specs=pl.BlockSpec((oc, MBLK, 128), lambda m, k: (0, m, 0)),
        out_shape=jax.ShapeDtypeStruct((oc, NPAD, 128), F32),
        scratch_shapes=[pltpu.VMEM((MBLK, m_out), F32)],
        compiler_params=pltpu.CompilerParams(
            dimension_semantics=("parallel", "arbitrary")),
    )(a1, w1, bias)


def _mm_add(a1, w1, rs, relu):
    nck = a1.shape[0]
    kc = 2 if nck % 2 == 0 else 1
    nk = nck // kc
    m_out = w1.shape[1]
    oc = m_out // 128
    nr = len(rs)

    def body(a1_ref, w1_ref, *rest):
        r_refs = rest[:nr]
        o_ref = rest[nr]
        acc = rest[nr + 1]
        ki = pl.program_id(1)

        @pl.when(ki == 0)
        def _():
            acc[...] = jnp.zeros_like(acc)

        a = jnp.concatenate([a1_ref[i] for i in range(kc)], axis=1) \
            if kc > 1 else a1_ref[0]
        acc[...] += jnp.dot(a, w1_ref[...], preferred_element_type=F32)

        @pl.when(ki == nk - 1)
        def _():
            for ci in range(oc):
                z = acc[:, ci * 128:(ci + 1) * 128]
                for r_ref in r_refs:
                    z = z + r_ref[ci]
                if relu:
                    z = jnp.maximum(z, 0.0)
                o_ref[ci] = z

    return pl.pallas_call(
        body,
        grid=(NBLK, nk),
        in_specs=[
            pl.BlockSpec((kc, MBLK, 128), lambda m, k: (k, m, 0)),
            pl.BlockSpec((kc * 128, m_out), lambda m, k: (k, 0)),
        ] + [pl.BlockSpec((oc, MBLK, 128), lambda m, k: (0, m, 0))] * nr,
        out_specs=pl.BlockSpec((oc, MBLK, 128), lambda m, k: (0, m, 0)),
        out_shape=jax.ShapeDtypeStruct((oc, NPAD, 128), F32),
        scratch_shapes=[pltpu.VMEM((MBLK, m_out), F32)],
        compiler_params=pltpu.CompilerParams(
            dimension_semantics=("parallel", "arbitrary")),
    )(a1, w1, *rs)


def _h2enc(a2, y2r2, batch3):
    def body(a_ref, r_ref, bt_ref, h_ref, e_ref, hs, cs):
        m = pl.program_id(0)

        @pl.when(m == 0)
        def _():
            hs[...] = jnp.zeros_like(hs)
            cs[...] = jnp.zeros_like(cs)

        h = jnp.maximum(a_ref[0] + a_ref[1] + r_ref[0], 0.0)
        h_ref[...] = h
        bt = bt_ref[0, 0]
        grp = lax.broadcasted_iota(jnp.int32, (G, MBLK), 0)
        onehot = (bt[None, :] == grp).astype(F32)
        hs[...] += jnp.dot(onehot, h, preferred_element_type=F32)
        cnt = jnp.sum(onehot, axis=1, keepdims=True)
        cs[...] += jnp.broadcast_to(cnt, (G, 128))

        @pl.when(m == NBLK - 1)
        def _():
            e_ref[...] = hs[...] / jnp.maximum(cs[...], 1.0)

    return pl.pallas_call(
        body,
        grid=(NBLK,),
        in_specs=[
            pl.BlockSpec((2, MBLK, 128), lambda m: (0, m, 0)),
            pl.BlockSpec((1, MBLK, 128), lambda m: (1, m, 0)),
            pl.BlockSpec((1, 1, MBLK), lambda m: (m, 0, 0)),
        ],
        out_specs=[
            pl.BlockSpec((MBLK, 128), lambda m: (m, 0)),
            pl.BlockSpec((G, 128), lambda m: (0, 0)),
        ],
        out_shape=[
            jax.ShapeDtypeStruct((NPAD, 128), F32),
            jax.ShapeDtypeStruct((G, 128), F32),
        ],
        scratch_shapes=[pltpu.VMEM((G, 128), F32), pltpu.VMEM((G, 128), F32)],
        compiler_params=pltpu.CompilerParams(
            dimension_semantics=("arbitrary",)),
    )(a2, y2r2, batch3)


def _final(a5, y5r5):
    def body(a_ref, r_ref, o_ref):
        o_ref[...] = a_ref[0] + a_ref[1] + r_ref[0]

    return pl.pallas_call(
        body,
        grid=(NBLK,),
        in_specs=[
            pl.BlockSpec((2, MBLK, 128), lambda m: (0, m, 0)),
            pl.BlockSpec((1, MBLK, 128), lambda m: (1, m, 0)),
        ],
        out_specs=pl.BlockSpec((MBLK, 128), lambda m: (m, 0)),
        out_shape=jax.ShapeDtypeStruct((NPAD, 128), F32),
    )(a5, y5r5)


def _padw(w, r, c):
    return jnp.pad(w, ((0, r - w.shape[0]), (0, c - w.shape[1])))


def _padb(b, m_out):
    return jnp.pad(b, (0, m_out - b.shape[0])).reshape(1, m_out)


def kernel(x, edge_index, edge_attr, batch,
           w0_rel, b0, w0_root, w1_rel, b1, w1_root, w2_rel, b2, w2_root,
           w3_rel, b3, w3_root, w4_rel, b4, w4_root, w5_rel, b5, w5_root):
    src = edge_index[0]
    dst = edge_index[1]
    pad_e = EPAD - E
    src_p = jnp.concatenate([src, jnp.zeros((pad_e,), jnp.int32)])
    dst_p = jnp.concatenate([dst, jnp.full((pad_e,), TRASH, jnp.int32)])
    src_all = (src_p[None, :]
               + (jnp.arange(HC, dtype=jnp.int32) * NPAD)[:, None]
               ).reshape(HC * EPAD // EB, EB)
    src_p = src_p.reshape(EPAD // EB, EB)
    dst_p = dst_p.reshape(EPAD // EB, EB)
    batch3 = jnp.concatenate(
        [batch, jnp.full((NPAD - N,), G, jnp.int32)]).reshape(NBLK, 1, MBLK)
    xp = jnp.pad(x, ((0, NPAD - N), (0, 0)))
    zrows = jnp.zeros((TPR, 128), F32)

    w0a = _padw(jnp.concatenate([w0_rel, w0_rel], axis=0), 256, HP)
    w0b = _padw(w0_root, 128, HP)
    bias0 = _padb(b0, HP)
    w1a = _padw(w1_rel, HP, HP)
    w1b = _padw(w1_root, HP, HP)
    bias1 = _padb(b1, HP)
    w2c = _padw(jnp.concatenate([w2_rel, w2_root], axis=1), HP, 256)
    bias2c = jnp.concatenate([jnp.zeros((128,), F32), b2]).reshape(1, 256)
    w3a = _padw(jnp.concatenate([w3_rel, w3_rel], axis=0), 256, HP)
    w3b = _padw(w3_root, 128, HP)
    bias3 = _padb(b3, HP)
    w4a = _padw(w4_rel, HP, HP)
    w4b = _padw(w4_root, HP, HP)
    bias4 = _padb(b4, HP)
    w5c = _padw(jnp.concatenate([w5_rel, w5_root], axis=1), HP, 256)
    bias5c = jnp.concatenate([jnp.zeros((128,), F32), b5]).reshape(1, 256)

    xc = xp.reshape(1, NPAD, 128)
    zbH = jnp.zeros((1, HP), F32)
    w1a_top, w1a_bot = w1a[:HP // 2], w1a[HP // 2:]
    w4a_top, w4a_bot = w4a[:HP // 2], w4a[HP // 2:]

    a0 = _seg128(xp, src_p, dst_p, zrows)
    r0 = _mm1(xc, w0b, bias0, False)
    h0 = _mm_add(a0, w0a, [r0], True)
    h0f = h0.reshape(HC * NPAD, 128)
    a1a = _segchunk(h0f, src_all, dst_p, zrows, 0, 8)
    r1 = _mm1(h0, w1b, bias1, False)
    t1 = _mm1(a1a, w1a_top, zbH, False)
    a1b = _segchunk(h0f, src_all, dst_p, zrows, 8, 8)
    h1 = _mm_add(a1b, w1a_bot, [t1, r1], True)
    y2r2 = _mm1(h1, w2c, bias2c, False)
    a2 = _seg128(y2r2.reshape(2 * NPAD, 128), src_p, dst_p, zrows)
    h2, enc = _h2enc(a2, y2r2, batch3)
    a3 = _seg128(h2, src_p, dst_p, zrows)
    r3 = _mm1(h2.reshape(1, NPAD, 128), w3b, bias3, False)
    h3 = _mm_add(a3, w3a, [r3], True)
    h3f = h3.reshape(HC * NPAD, 128)
    a4a = _segchunk(h3f, src_all, dst_p, zrows, 0, 8)
    r4 = _mm1(h3, w4b, bias4, False)
    t4 = _mm1(a4a, w4a_top, zbH, False)
    a4b = _segchunk(h3f, src_all, dst_p, zrows, 8, 8)
    h4 = _mm_add(a4b, w4a_bot, [t4, r4], True)
    y5r5 = _mm1(h4, w5c, bias5c, False)
    a5 = _seg128(y5r5.reshape(2 * NPAD, 128), src_p, dst_p, zrows)
    out = _final(a5, y5r5)
    return (out[:N], enc)


# speedup vs baseline: 3.5286x; 1.0223x over previous
import functools

import jax
import jax.numpy as jnp
from jax import lax
from jax.experimental import pallas as pl
from jax.experimental.pallas import tpu as pltpu
from jax.experimental.pallas import tpu_sc as plsc

N = 10000
E = 320000
G = 16
NPAD = 10240
TRASH = N
MBLK = 256
NBLK = NPAD // MBLK
EB = 128
EPAD = 327680
SROWS = 10240
TPR = SROWS // 16
HP = 2048
HC = HP // 128
F32 = jnp.float32


def _seg_mesh():
    return plsc.VectorSubcoreMesh(core_axis_name="c", subcore_axis_name="s",
                                  num_cores=2, num_subcores=16)


NRING = 2
IB = 40


def _edge_pipeline(table_r, shared, src_hbm, src_base, dst_hbm, dst_base,
                   src_v, dst_v, rows, gsems, ssems, nb):
    def gather(i, b):
        pltpu.async_copy(table_r.at[src_v.at[i]], rows.at[b], gsems[b])

    def wait_gather(b):
        pltpu.make_async_copy(table_r.at[src_v.at[0]], rows.at[b],
                              gsems[b]).wait()

    def scatter(i, b):
        pltpu.async_copy(rows.at[b], shared.at[dst_v.at[i]], ssems[b],
                         add=True)

    def wait_scatter(b):
        pltpu.make_async_copy(rows.at[b], shared.at[dst_v.at[0]],
                              ssems[b]).wait()

    def block(j2, carry):
        pltpu.sync_copy(src_hbm.at[pl.ds(src_base + j2 * IB, IB)], src_v)
        pltpu.sync_copy(dst_hbm.at[pl.ds(dst_base + j2 * IB, IB)], dst_v)
        for b in range(NRING):
            gather(b, b)

        def body(i, c2):
            for b in range(NRING):
                wait_gather(b)
                scatter(i + b, b)
            for b in range(NRING):
                wait_scatter(b)

                @pl.when(i + NRING + b < IB)
                def _():
                    gather(i + NRING + b, b)

            return c2

        lax.fori_loop(0, IB // NRING, lambda i2, c2: body(i2 * NRING, c2), 0)
        return carry

    lax.fori_loop(0, nb // IB, block, 0)


def _seg128(table, src2d, dst2d, zrows):
    ept = EPAD // 32
    nb = ept // EB

    @functools.partial(
        pl.kernel,
        mesh=_seg_mesh(),
        out_type=jax.ShapeDtypeStruct((2 * NPAD, 128), F32),
        scratch_types=[
            pltpu.VMEM((IB, EB), jnp.int32),
            pltpu.VMEM((IB, EB), jnp.int32),
            pltpu.VMEM((NRING, EB, 128), F32),
            pltpu.VMEM_SHARED((SROWS, 128), F32),
        ] + [pltpu.SemaphoreType.DMA] * (2 * NRING),
    )
    def k(table_r, src_r, dst_r, z_r, out_r, src_v, dst_v, rows, shared,
          *sems):
        c = lax.axis_index("c")
        s = lax.axis_index("s")
        g = c * 16 + s
        pltpu.sync_copy(z_r, shared.at[pl.ds(s * TPR, TPR)])
        plsc.subcore_barrier()
        _edge_pipeline(table_r, shared, src_r, g * nb, dst_r, g * nb,
                       src_v, dst_v, rows, sems[:NRING], sems[NRING:], nb)
        plsc.subcore_barrier()
        pltpu.sync_copy(
            shared.at[pl.ds(s * TPR, TPR)],
            out_r.at[pl.ds(c * NPAD + s * TPR, TPR)],
        )

    return k(table, src2d, dst2d, zrows).reshape(2, NPAD, 128)


def _segchunk(table, src_all, dst2d, zrows, base, nch):
    ept = EPAD // 16
    nb = ept // EB
    nbt = EPAD // EB
    cpc = nch // 2

    @functools.partial(
        pl.kernel,
        mesh=_seg_mesh(),
        out_type=jax.ShapeDtypeStruct((nch * NPAD, 128), F32),
        scratch_types=[
            pltpu.VMEM((IB, EB), jnp.int32),
            pltpu.VMEM((IB, EB), jnp.int32),
            pltpu.VMEM((NRING, EB, 128), F32),
            pltpu.VMEM_SHARED((SROWS, 128), F32),
        ] + [pltpu.SemaphoreType.DMA] * (2 * NRING),
    )
    def k(table_r, src_r, dst_r, z_r, out_r, src_v, dst_v, rows, shared,
          *sems):
        c = lax.axis_index("c")
        s = lax.axis_index("s")

        def chunk_body(j, carry):
            ch = base + c * cpc + j
            och = c * cpc + j
            pltpu.sync_copy(z_r, shared.at[pl.ds(s * TPR, TPR)])
            plsc.subcore_barrier()
            _edge_pipeline(table_r, shared, src_r, ch * nbt + s * nb,
                           dst_r, s * nb, src_v, dst_v, rows,
                           sems[:NRING], sems[NRING:], nb)
            plsc.subcore_barrier()
            pltpu.sync_copy(
                shared.at[pl.ds(s * TPR, TPR)],
                out_r.at[pl.ds(och * NPAD + s * TPR, TPR)],
            )
            return carry

        lax.fori_loop(0, cpc, chunk_body, 0)

    return k(table, src_all, dst2d, zrows).reshape(nch, NPAD, 128)


def _mm1(a1, w1, bias, relu):
    nck = a1.shape[0]
    kc = 4 if nck % 4 == 0 else (2 if nck % 2 == 0 else 1)
    nk = nck // kc
    m_out = w1.shape[1]
    oc = m_out // 128

    def body(a1_ref, w1_ref, b_ref, o_ref, acc):
        ki = pl.program_id(1)

        @pl.when(ki == 0)
        def _():
            acc[...] = jnp.zeros_like(acc)

        a = jnp.concatenate([a1_ref[i] for i in range(kc)], axis=1) \
            if kc > 1 else a1_ref[0]
        acc[...] += jnp.dot(a, w1_ref[...], preferred_element_type=F32)

        @pl.when(ki == nk - 1)
        def _():
            z = acc[...] + b_ref[...]
            if relu:
                z = jnp.maximum(z, 0.0)
            for ci in range(oc):
                o_ref[ci] = z[:, ci * 128:(ci + 1) * 128]

    return pl.pallas_call(
        body,
        grid=(NBLK, nk),
        in_specs=[
            pl.BlockSpec((kc, MBLK, 128), lambda m, k: (k, m, 0)),
            pl.BlockSpec((kc * 128, m_out), lambda m, k: (k, 0)),
            pl.BlockSpec((1, m_out), lambda m, k: (0, 0)),
        ],
        out_specs=pl.BlockSpec((oc, MBLK, 128), lambda m, k: (0, m, 0)),
        out_shape=jax.ShapeDtypeStruct((oc, NPAD, 128), F32),
        scratch_shapes=[pltpu.VMEM((MBLK, m_out), F32)],
        compiler_params=pltpu.CompilerParams(
            dimension_semantics=("parallel", "arbitrary")),
    )(a1, w1, bias)


def _mm_add(a1, w1, rs, relu):
    nck = a1.shape[0]
    kc = 4 if nck % 4 == 0 else (2 if nck % 2 == 0 else 1)
    nk = nck // kc
    m_out = w1.shape[1]
    oc = m_out // 128
    nr = len(rs)

    def body(a1_ref, w1_ref, *rest):
        r_refs = rest[:nr]
        o_ref = rest[nr]
        acc = rest[nr + 1]
        ki = pl.program_id(1)

        @pl.when(ki == 0)
        def _():
            acc[...] = jnp.zeros_like(acc)

        a = jnp.concatenate([a1_ref[i] for i in range(kc)], axis=1) \
            if kc > 1 else a1_ref[0]
        acc[...] += jnp.dot(a, w1_ref[...], preferred_element_type=F32)

        @pl.when(ki == nk - 1)
        def _():
            for ci in range(oc):
                z = acc[:, ci * 128:(ci + 1) * 128]
                for r_ref in r_refs:
                    z = z + r_ref[ci]
                if relu:
                    z = jnp.maximum(z, 0.0)
                o_ref[ci] = z

    return pl.pallas_call(
        body,
        grid=(NBLK, nk),
        in_specs=[
            pl.BlockSpec((kc, MBLK, 128), lambda m, k: (k, m, 0)),
            pl.BlockSpec((kc * 128, m_out), lambda m, k: (k, 0)),
        ] + [pl.BlockSpec((oc, MBLK, 128), lambda m, k: (0, m, 0))] * nr,
        out_specs=pl.BlockSpec((oc, MBLK, 128), lambda m, k: (0, m, 0)),
        out_shape=jax.ShapeDtypeStruct((oc, NPAD, 128), F32),
        scratch_shapes=[pltpu.VMEM((MBLK, m_out), F32)],
        compiler_params=pltpu.CompilerParams(
            dimension_semantics=("parallel", "arbitrary")),
    )(a1, w1, *rs)


def _h2enc(a2, y2r2, batch3):
    def body(a_ref, r_ref, bt_ref, h_ref, e_ref, hs, cs):
        m = pl.program_id(0)

        @pl.when(m == 0)
        def _():
            hs[...] = jnp.zeros_like(hs)
            cs[...] = jnp.zeros_like(cs)

        h = jnp.maximum(a_ref[0] + a_ref[1] + r_ref[0], 0.0)
        h_ref[...] = h
        bt = bt_ref[0, 0]
        grp = lax.broadcasted_iota(jnp.int32, (G, MBLK), 0)
        onehot = (bt[None, :] == grp).astype(F32)
        hs[...] += jnp.dot(onehot, h, preferred_element_type=F32)
        cnt = jnp.sum(onehot, axis=1, keepdims=True)
        cs[...] += jnp.broadcast_to(cnt, (G, 128))

        @pl.when(m == NBLK - 1)
        def _():
            e_ref[...] = hs[...] / jnp.maximum(cs[...], 1.0)

    return pl.pallas_call(
        body,
        grid=(NBLK,),
        in_specs=[
            pl.BlockSpec((2, MBLK, 128), lambda m: (0, m, 0)),
            pl.BlockSpec((1, MBLK, 128), lambda m: (1, m, 0)),
            pl.BlockSpec((1, 1, MBLK), lambda m: (m, 0, 0)),
        ],
        out_specs=[
            pl.BlockSpec((MBLK, 128), lambda m: (m, 0)),
            pl.BlockSpec((G, 128), lambda m: (0, 0)),
        ],
        out_shape=[
            jax.ShapeDtypeStruct((NPAD, 128), F32),
            jax.ShapeDtypeStruct((G, 128), F32),
        ],
        scratch_shapes=[pltpu.VMEM((G, 128), F32), pltpu.VMEM((G, 128), F32)],
        compiler_params=pltpu.CompilerParams(
            dimension_semantics=("arbitrary",)),
    )(a2, y2r2, batch3)


def _final(a5, y5r5):
    def body(a_ref, r_ref, o_ref):
        o_ref[...] = a_ref[0] + a_ref[1] + r_ref[0]

    return pl.pallas_call(
        body,
        grid=(NBLK,),
        in_specs=[
            pl.BlockSpec((2, MBLK, 128), lambda m: (0, m, 0)),
            pl.BlockSpec((1, MBLK, 128), lambda m: (1, m, 0)),
        ],
        out_specs=pl.BlockSpec((MBLK, 128), lambda m: (m, 0)),
        out_shape=jax.ShapeDtypeStruct((NPAD, 128), F32),
    )(a5, y5r5)


def _padw(w, r, c):
    return jnp.pad(w, ((0, r - w.shape[0]), (0, c - w.shape[1])))


def _padb(b, m_out):
    return jnp.pad(b, (0, m_out - b.shape[0])).reshape(1, m_out)


def kernel(x, edge_index, edge_attr, batch,
           w0_rel, b0, w0_root, w1_rel, b1, w1_root, w2_rel, b2, w2_root,
           w3_rel, b3, w3_root, w4_rel, b4, w4_root, w5_rel, b5, w5_root):
    src = edge_index[0]
    dst = edge_index[1]
    pad_e = EPAD - E
    src_p = jnp.concatenate([src, jnp.zeros((pad_e,), jnp.int32)])
    dst_p = jnp.concatenate([dst, jnp.full((pad_e,), TRASH, jnp.int32)])
    src_all = (src_p[None, :]
               + (jnp.arange(HC, dtype=jnp.int32) * NPAD)[:, None]
               ).reshape(HC * EPAD // EB, EB)
    src_p = src_p.reshape(EPAD // EB, EB)
    dst_p = dst_p.reshape(EPAD // EB, EB)
    batch3 = jnp.concatenate(
        [batch, jnp.full((NPAD - N,), G, jnp.int32)]).reshape(NBLK, 1, MBLK)
    xp = jnp.pad(x, ((0, NPAD - N), (0, 0)))
    zrows = jnp.zeros((TPR, 128), F32)

    w0a = _padw(jnp.concatenate([w0_rel, w0_rel], axis=0), 256, HP)
    w0b = _padw(w0_root, 128, HP)
    bias0 = _padb(b0, HP)
    w1a = _padw(w1_rel, HP, HP)
    w1b = _padw(w1_root, HP, HP)
    bias1 = _padb(b1, HP)
    w2c = _padw(jnp.concatenate([w2_rel, w2_root], axis=1), HP, 256)
    bias2c = jnp.concatenate([jnp.zeros((128,), F32), b2]).reshape(1, 256)
    w3a = _padw(jnp.concatenate([w3_rel, w3_rel], axis=0), 256, HP)
    w3b = _padw(w3_root, 128, HP)
    bias3 = _padb(b3, HP)
    w4a = _padw(w4_rel, HP, HP)
    w4b = _padw(w4_root, HP, HP)
    bias4 = _padb(b4, HP)
    w5c = _padw(jnp.concatenate([w5_rel, w5_root], axis=1), HP, 256)
    bias5c = jnp.concatenate([jnp.zeros((128,), F32), b5]).reshape(1, 256)

    xc = xp.reshape(1, NPAD, 128)
    zbH = jnp.zeros((1, HP), F32)
    w1a_top, w1a_bot = w1a[:HP // 2], w1a[HP // 2:]
    w4a_top, w4a_bot = w4a[:HP // 2], w4a[HP // 2:]

    a0 = _seg128(xp, src_p, dst_p, zrows)
    r0 = _mm1(xc, w0b, bias0, False)
    h0 = _mm_add(a0, w0a, [r0], True)
    h0f = h0.reshape(HC * NPAD, 128)
    a1a = _segchunk(h0f, src_all, dst_p, zrows, 0, 8)
    r1 = _mm1(h0, w1b, bias1, False)
    t1 = _mm1(a1a, w1a_top, zbH, False)
    a1b = _segchunk(h0f, src_all, dst_p, zrows, 8, 8)
    h1 = _mm_add(a1b, w1a_bot, [t1, r1], True)
    y2r2 = _mm1(h1, w2c, bias2c, False)
    a2 = _seg128(y2r2.reshape(2 * NPAD, 128), src_p, dst_p, zrows)
    h2, enc = _h2enc(a2, y2r2, batch3)
    a3 = _seg128(h2, src_p, dst_p, zrows)
    r3 = _mm1(h2.reshape(1, NPAD, 128), w3b, bias3, False)
    h3 = _mm_add(a3, w3a, [r3], True)
    h3f = h3.reshape(HC * NPAD, 128)
    a4a = _segchunk(h3f, src_all, dst_p, zrows, 0, 8)
    r4 = _mm1(h3, w4b, bias4, False)
    t4 = _mm1(a4a, w4a_top, zbH, False)
    a4b = _segchunk(h3f, src_all, dst_p, zrows, 8, 8)
    h4 = _mm_add(a4b, w4a_bot, [t4, r4], True)
    y5r5 = _mm1(h4, w5c, bias5c, False)
    a5 = _seg128(y5r5.reshape(2 * NPAD, 128), src_p, dst_p, zrows)
    out = _final(a5, y5r5)
    return (out[:N], enc)
